# baseline (speedup 1.0000x reference)
import jax
import jax.numpy as jnp
from jax.experimental import pallas as pl
from jax.experimental.pallas import tpu as pltpu

_CHUNKS = 4


def _copy_body(se_in, rw_in, se_out, rw_out, se_b0, se_b1, rw_b0, rw_b1,
               in_s0, in_s1, out_s0, out_s1):
    rows = se_out.shape[1]
    c = rows // _CHUNKS
    se_bufs = (se_b0, se_b1)
    rw_bufs = (rw_b0, rw_b1)
    in_sems = (in_s0, in_s1)
    out_sems = (out_s0, out_s1)

    def in_copy(i):
        b = i % 2
        sl = pl.ds(i * c, c)
        return (pltpu.make_async_copy(se_in.at[:, sl], se_bufs[b], in_sems[b]),
                pltpu.make_async_copy(rw_in.at[:, sl], rw_bufs[b], in_sems[b]))

    def out_copy(i):
        b = i % 2
        sl = pl.ds(i * c, c)
        return (pltpu.make_async_copy(se_bufs[b], se_out.at[:, sl], out_sems[b]),
                pltpu.make_async_copy(rw_bufs[b], rw_out.at[:, sl], out_sems[b]))

    pending_out = [None, None]
    pending_in = [None, None]
    pending_in[0] = in_copy(0)
    for a in pending_in[0]:
        a.start()
    for i in range(_CHUNKS):
        b = i % 2
        nxt = i + 1
        if nxt < _CHUNKS:
            nb = nxt % 2
            if pending_out[nb] is not None:
                for a in pending_out[nb]:
                    a.wait()
                pending_out[nb] = None
            pending_in[nb] = in_copy(nxt)
            for a in pending_in[nb]:
                a.start()
        for a in pending_in[b]:
            a.wait()
        pending_in[b] = None
        pending_out[b] = out_copy(i)
        for a in pending_out[b]:
            a.start()
    for p in pending_out:
        if p is not None:
            for a in p:
                a.wait()


def kernel(hidden_states, selected_experts, routing_weights):
    rows = hidden_states.shape[0] * hidden_states.shape[1]
    k = selected_experts.shape[1]
    out_dtype = hidden_states.dtype

    se_t = selected_experts.T
    rw_t = routing_weights.astype(out_dtype).T
    c = rows // _CHUNKS

    se_o, rw_o = pl.pallas_call(
        _copy_body,
        in_specs=[
            pl.BlockSpec(memory_space=pl.ANY),
            pl.BlockSpec(memory_space=pl.ANY),
        ],
        out_specs=[
            pl.BlockSpec(memory_space=pl.ANY),
            pl.BlockSpec(memory_space=pl.ANY),
        ],
        out_shape=[
            jax.ShapeDtypeStruct((k, rows), selected_experts.dtype),
            jax.ShapeDtypeStruct((k, rows), out_dtype),
        ],
        scratch_shapes=[
            pltpu.VMEM((k, c), selected_experts.dtype),
            pltpu.VMEM((k, c), selected_experts.dtype),
            pltpu.VMEM((k, c), out_dtype),
            pltpu.VMEM((k, c), out_dtype),
            pltpu.SemaphoreType.DMA,
            pltpu.SemaphoreType.DMA,
            pltpu.SemaphoreType.DMA,
            pltpu.SemaphoreType.DMA,
        ],
    )(se_t, rw_t)

    return se_o.T, rw_o.T


# speedup vs baseline: 1.5978x; 1.5978x over previous
import jax
import jax.numpy as jnp
from jax.experimental import pallas as pl
from jax.experimental.pallas import tpu as pltpu

_CHUNKS = 2


def _copy_body(se_in, rw_in, se_out, rw_out, se_b0, se_b1, rw_b0, rw_b1,
               in_s0, in_s1, out_s0, out_s1):
    rows = se_out.shape[1]
    c = rows // _CHUNKS
    se_bufs = (se_b0, se_b1)
    rw_bufs = (rw_b0, rw_b1)
    in_sems = (in_s0, in_s1)
    out_sems = (out_s0, out_s1)

    def in_copy(i):
        b = i % 2
        sl = pl.ds(i * c, c)
        return (pltpu.make_async_copy(se_in.at[:, sl], se_bufs[b], in_sems[b]),
                pltpu.make_async_copy(rw_in.at[:, sl], rw_bufs[b], in_sems[b]))

    def out_copy(i):
        b = i % 2
        sl = pl.ds(i * c, c)
        return (pltpu.make_async_copy(se_bufs[b], se_out.at[:, sl], out_sems[b]),
                pltpu.make_async_copy(rw_bufs[b], rw_out.at[:, sl], out_sems[b]))

    pending_out = [None, None]
    pending_in = [None, None]
    pending_in[0] = in_copy(0)
    for a in pending_in[0]:
        a.start()
    for i in range(_CHUNKS):
        b = i % 2
        nxt = i + 1
        if nxt < _CHUNKS:
            nb = nxt % 2
            if pending_out[nb] is not None:
                for a in pending_out[nb]:
                    a.wait()
                pending_out[nb] = None
            pending_in[nb] = in_copy(nxt)
            for a in pending_in[nb]:
                a.start()
        for a in pending_in[b]:
            a.wait()
        pending_in[b] = None
        pending_out[b] = out_copy(i)
        for a in pending_out[b]:
            a.start()
    for p in pending_out:
        if p is not None:
            for a in p:
                a.wait()


def kernel(hidden_states, selected_experts, routing_weights):
    rows = hidden_states.shape[0] * hidden_states.shape[1]
    k = selected_experts.shape[1]
    out_dtype = hidden_states.dtype

    se_t = selected_experts.T
    rw_t = routing_weights.astype(out_dtype).T
    c = rows // _CHUNKS

    se_o, rw_o = pl.pallas_call(
        _copy_body,
        in_specs=[
            pl.BlockSpec(memory_space=pl.ANY),
            pl.BlockSpec(memory_space=pl.ANY),
        ],
        out_specs=[
            pl.BlockSpec(memory_space=pl.ANY),
            pl.BlockSpec(memory_space=pl.ANY),
        ],
        out_shape=[
            jax.ShapeDtypeStruct((k, rows), selected_experts.dtype),
            jax.ShapeDtypeStruct((k, rows), out_dtype),
        ],
        scratch_shapes=[
            pltpu.VMEM((k, c), selected_experts.dtype),
            pltpu.VMEM((k, c), selected_experts.dtype),
            pltpu.VMEM((k, c), out_dtype),
            pltpu.VMEM((k, c), out_dtype),
            pltpu.SemaphoreType.DMA,
            pltpu.SemaphoreType.DMA,
            pltpu.SemaphoreType.DMA,
            pltpu.SemaphoreType.DMA,
        ],
    )(se_t, rw_t)

    return se_o.T, rw_o.T


# speedup vs baseline: 1.6342x; 1.0228x over previous
import jax
import jax.numpy as jnp
from jax.experimental import pallas as pl


_GRID = 2


def _copy_body(se_in, rw_in, se_out, rw_out):
    se_out[...] = se_in[...]
    rw_out[...] = rw_in[...]


def kernel(hidden_states, selected_experts, routing_weights):
    rows = hidden_states.shape[0] * hidden_states.shape[1]
    k = selected_experts.shape[1]
    out_dtype = hidden_states.dtype

    se_t = selected_experts.T
    rw_t = routing_weights.astype(out_dtype).T

    block = rows // _GRID
    spec = pl.BlockSpec((k, block), lambda i: (0, i))

    se_o, rw_o = pl.pallas_call(
        _copy_body,
        grid=(_GRID,),
        in_specs=[spec, spec],
        out_specs=[spec, spec],
        out_shape=[
            jax.ShapeDtypeStruct((k, rows), selected_experts.dtype),
            jax.ShapeDtypeStruct((k, rows), out_dtype),
        ],
    )(se_t, rw_t)

    return se_o.T, rw_o.T


# speedup vs baseline: 1.6364x; 1.0014x over previous
import jax
from jax.experimental import pallas as pl


_GRID = 2


def _copy_body(se_in, rw_in, se_out, rw_out):
    se_out[...] = se_in[...]
    rw_out[...] = rw_in[...]


def kernel(hidden_states, selected_experts, routing_weights):
    rows = hidden_states.shape[0] * hidden_states.shape[1]
    k = selected_experts.shape[1]
    out_dtype = hidden_states.dtype

    se_t = selected_experts.T
    rw_t = routing_weights.astype(out_dtype).T

    block = rows // _GRID
    spec = pl.BlockSpec((k, block), lambda i: (0, i))

    se_o, rw_o = pl.pallas_call(
        _copy_body,
        grid=(_GRID,),
        in_specs=[spec, spec],
        out_specs=[spec, spec],
        out_shape=[
            jax.ShapeDtypeStruct((k, rows), selected_experts.dtype),
            jax.ShapeDtypeStruct((k, rows), out_dtype),
        ],
    )(se_t, rw_t)

    return se_o.T, rw_o.T
